# baseline (speedup 1.0000x reference)
import jax
import jax.numpy as jnp
from jax import lax
from jax.experimental import pallas as pl
from jax.experimental.pallas import tpu as pltpu
from jax.experimental.pallas import tpu_sc as plsc

f32 = jnp.float32
i32 = jnp.int32

N = 10000
E = 320000
NP = 10112
NE2 = 2 * NP
LENE = 327680
GROUP = 40
DEPTH1 = 4
DEPTH2 = 8
STR1 = NE2 // 16
STR2 = NP // 16


def _mesh():
    return plsc.VectorSubcoreMesh(
        core_axis_name="c", subcore_axis_name="s", num_cores=2,
        num_subcores=16)


_SC_PARAMS = dict(compiler_params=pltpu.CompilerParams(
    use_tc_tiling_on_sc=False))


def _mm_body(xb, wb, ob):
    ob[...] = jnp.dot(xb[...], wb[...], preferred_element_type=f32)


def _tc_matmul(xp, wcat):
    return pl.pallas_call(
        _mm_body,
        grid=(NP // 128,),
        in_specs=[pl.BlockSpec((128, 128), lambda i: (i, 0)),
                  pl.BlockSpec((128, 384), lambda i: (0, 0))],
        out_specs=pl.BlockSpec((128, 384), lambda i: (i, 0)),
        out_shape=jax.ShapeDtypeStruct((NP, 384), f32),
    )(xp, wcat)


def _idx_body(sb, db, rb, es, ed, ss, dd):
    sv, dv, rv = sb[...], db[...], rb[...]
    flat = (lax.broadcasted_iota(i32, (LENE // 128, 128), 0) * 128
            + lax.broadcasted_iota(i32, (LENE // 128, 128), 1))
    valid = flat < E
    low = lax.bitwise_and(flat, 63)
    es[...] = jnp.where(valid, sv * 2 + rv, low)
    ed[...] = jnp.where(valid, dv * 2 + rv, 2 * N + low)
    ss[...] = jnp.where(valid, sv, low)
    dd[...] = jnp.where(valid, dv, N + low)


def _tc_indices(srcm, dstm, revm):
    spec = pl.BlockSpec((LENE // 128, 128), lambda i: (0, 0))
    sh = jax.ShapeDtypeStruct((LENE // 128, 128), i32)
    return pl.pallas_call(
        _idx_body,
        grid=(1,),
        in_specs=[spec, spec, spec],
        out_specs=[spec, spec, spec, spec],
        out_shape=[sh, sh, sh, sh],
    )(srcm, dstm, revm)


def _tab_body(cb, hb, t0, t1, t2, t3, db):
    c = cb[0] + cb[1]
    cst, cts = c[:, 0, :], c[:, 1, :]
    d_st = lax.rsqrt(cst + 1.0)
    d_ts = lax.rsqrt(cts + 1.0)
    d_al = lax.rsqrt(cst + cts + 1.0)
    db[...] = jnp.concatenate(
        [d_st[:, None, :], d_al[:, None, :], d_ts[:, None, :]], axis=1)
    gst = d_st * hb[:, 0, :]
    gall = d_al * hb[:, 1, :]
    gts = d_ts * hb[:, 2, :]
    for g, tr in enumerate((t0, t1, t2, t3)):
        cg = slice(32 * g, 32 * g + 32)
        ra = jnp.concatenate([gst[:, cg], gall[:, cg]], axis=1)[:, None, :]
        rb = jnp.concatenate([gts[:, cg], gall[:, cg]], axis=1)[:, None, :]
        tr[...] = jnp.concatenate([ra, rb], axis=1)


def _tc_tables(cnt3, h33):
    sht = jax.ShapeDtypeStruct((NP, 2, 64), f32)
    return pl.pallas_call(
        _tab_body,
        grid=(NP // 128,),
        in_specs=[pl.BlockSpec((2, 128, 2, 1), lambda i: (0, i, 0, 0)),
                  pl.BlockSpec((128, 3, 128), lambda i: (i, 0, 0))],
        out_specs=[pl.BlockSpec((128, 2, 64), lambda i: (i, 0, 0))] * 4
        + [pl.BlockSpec((128, 3, 1), lambda i: (i, 0, 0))],
        out_shape=[sht] * 4 + [jax.ShapeDtypeStruct((NP, 3, 1), f32)],
    )(cnt3, h33)


def _l2_body(m00, m01, m02, m03, m10, m11, m12, m13, hb, db, bcb, wb, pb, qb):
    m0s, m1s = (m00, m01, m02, m03), (m10, m11, m12, m13)
    st, ts, al = [], [], []
    for g in range(4):
        sp = m0s[g][...] + m1s[g][...]
        st.append(sp[:, 0, 0:32])
        ts.append(sp[:, 1, 0:32])
        al.append(sp[:, 0, 32:64] + sp[:, 1, 32:64])
    aggs = (jnp.concatenate(st, axis=1), jnp.concatenate(al, axis=1),
            jnp.concatenate(ts, axis=1))
    acc = jnp.zeros((128, 48), f32)
    for c in range(3):
        dd = db[:, c, :]
        hc = jnp.maximum(
            dd * aggs[c] + dd * dd * hb[:, c, :] + bcb[c][None, :], 0.0)
        acc = acc + jnp.dot(hc, wb[c], preferred_element_type=f32)
    pb[...] = acc
    qb[...] = db[:, 1, :] * acc


def _tc_l2(parts1, h33, dis3, bcat, w2p):
    sh48 = jax.ShapeDtypeStruct((NP, 48), f32)
    return pl.pallas_call(
        _l2_body,
        grid=(NP // 128,),
        in_specs=[pl.BlockSpec((128, 2, 64), lambda i: (i, 0, 0))] * 8
        + [pl.BlockSpec((128, 3, 128), lambda i: (i, 0, 0)),
           pl.BlockSpec((128, 3, 1), lambda i: (i, 0, 0)),
           pl.BlockSpec((3, 128), lambda i: (0, 0)),
           pl.BlockSpec((3, 128, 48), lambda i: (0, 0, 0))],
        out_specs=[pl.BlockSpec((128, 48), lambda i: (i, 0)),
                   pl.BlockSpec((128, 48), lambda i: (i, 0))],
        out_shape=[sh48, sh48],
    )(*parts1, h33, dis3, bcat, w2p)


def _out_body(ptb, pb, db, b2b, ob):
    s = ptb[0] + ptb[1]
    dd = db[...]
    o = dd * s[:, :40] + dd * dd * pb[:, :40] + b2b[...]
    m = jnp.max(o, axis=1, keepdims=True)
    z = jnp.sum(jnp.exp(o - m), axis=1, keepdims=True)
    ob[...] = o - m - jnp.log(z)


def _tc_out(parts, p, disall, b2p):
    return pl.pallas_call(
        _out_body,
        grid=(N // 80,),
        in_specs=[pl.BlockSpec((2, 80, 48), lambda i: (0, i, 0)),
                  pl.BlockSpec((80, 48), lambda i: (i, 0)),
                  pl.BlockSpec((80, 1), lambda i: (i, 0)),
                  pl.BlockSpec((1, 40), lambda i: (0, 0))],
        out_specs=pl.BlockSpec((80, 40), lambda i: (i, 0)),
        out_shape=jax.ShapeDtypeStruct((N, 40), f32),
    )(parts, p, disall, b2p)


def _hist_body(e2d, ones_h, z_h, out, idxd, ones_v, stage, hist, sem):
    cid = lax.axis_index("c")
    sid = lax.axis_index("s")
    pltpu.sync_copy(z_h, stage)
    pltpu.sync_copy(stage, hist.at[pl.ds(sid * STR1, STR1)])
    pltpu.sync_copy(ones_h, ones_v)
    plsc.subcore_barrier()
    base = (cid * 16 + sid) * 80
    for g in range(2):
        pltpu.sync_copy(e2d.at[pl.ds(base + g * GROUP, GROUP)], idxd)

        def it_body(it, carry):
            for b in range(8):
                pltpu.async_copy(ones_v, hist.at[idxd.at[it * 8 + b]], sem,
                                 add=True)
            for b in range(8):
                pltpu.make_async_copy(ones_v, hist.at[idxd.at[0]], sem).wait()
            return carry

        lax.fori_loop(0, 5, it_body, 0)
    plsc.subcore_barrier()
    pltpu.sync_copy(hist.at[pl.ds(sid * STR1, STR1)], stage)
    pltpu.sync_copy(stage, out.at[cid, sid])


def _sc_hist(e2d, ones_h, z_h):
    return pl.kernel(
        _hist_body,
        out_type=jax.ShapeDtypeStruct((2, 16, STR1), f32),
        mesh=_mesh(),
        scratch_types=[
            pltpu.VMEM((GROUP, 128), i32),
            pltpu.VMEM((128,), f32),
            pltpu.VMEM((STR1,), f32),
            pltpu.VMEM_SHARED((NE2,), f32),
            pltpu.SemaphoreType.DMA,
        ],
        **_SC_PARAMS,
    )(e2d, ones_h, z_h)


def _row_pipeline(src_hbm, dst_hbm, table, acc, idxg, idxd, rows, sem_g,
                  sem_s, base, groups, depth):
    for g in range(groups):
        if g > 0:
            for b in range(depth):
                pltpu.make_async_copy(
                    rows.at[b], acc.at[idxd.at[0]], sem_s.at[b]).wait()
        pltpu.sync_copy(src_hbm.at[pl.ds(base + g * GROUP, GROUP)], idxg)
        pltpu.sync_copy(dst_hbm.at[pl.ds(base + g * GROUP, GROUP)], idxd)

        def it_body(it, carry):
            for b in range(depth):
                @pl.when(it > 0)
                def _drain(b=b):
                    pltpu.make_async_copy(
                        rows.at[b], acc.at[idxd.at[0]], sem_s.at[b]).wait()
                pltpu.async_copy(
                    table.at[idxg.at[it * depth + b]], rows.at[b],
                    sem_g.at[b])
            for b in range(depth):
                pltpu.make_async_copy(
                    table.at[idxg.at[0]], rows.at[b], sem_g.at[b]).wait()
                pltpu.async_copy(
                    rows.at[b], acc.at[idxd.at[it * depth + b]],
                    sem_s.at[b], add=True)
            return carry

        lax.fori_loop(0, GROUP // depth, it_body, 0)
    for b in range(depth):
        pltpu.make_async_copy(rows.at[b], acc.at[idxd.at[0]], sem_s.at[b]).wait()


def _agg1_body(e2s, e2d, t0, t1, t2, t3, z64, out, idxg, idxd, rows, acc,
               sem_g, sem_s):
    cid = lax.axis_index("c")
    sid = lax.axis_index("s")
    r0 = sid * STR1
    base = cid * 1280 + sid * 80
    for p, tbl in enumerate((t0, t1, t2, t3)):
        pltpu.sync_copy(z64, rows.at[0])
        for k in range(9):
            pltpu.sync_copy(rows.at[0], acc.at[pl.ds(r0 + k * 128, 128)])
        pltpu.sync_copy(rows.at[0, pl.ds(0, 112)],
                        acc.at[pl.ds(r0 + 1152, 112)])
        plsc.subcore_barrier()
        _row_pipeline(e2s, e2d, tbl, acc, idxg, idxd, rows, sem_g, sem_s,
                      base, 2, DEPTH1)
        plsc.subcore_barrier()
        for k in range(9):
            pltpu.sync_copy(acc.at[pl.ds(r0 + k * 128, 128)], rows.at[0])
            pltpu.sync_copy(rows.at[0], out.at[cid, p, pl.ds(r0 + k * 128, 128)])
        pltpu.sync_copy(acc.at[pl.ds(r0 + 1152, 112)],
                        rows.at[0, pl.ds(0, 112)])
        pltpu.sync_copy(rows.at[0, pl.ds(0, 112)],
                        out.at[cid, p, pl.ds(r0 + 1152, 112)])
        plsc.subcore_barrier()


def _sc_agg1(e2s, e2d, t0, t1, t2, t3, z64):
    return pl.kernel(
        _agg1_body,
        out_type=jax.ShapeDtypeStruct((2, 4, NE2, 64), f32),
        mesh=_mesh(),
        scratch_types=[
            pltpu.VMEM((GROUP, 128), i32),
            pltpu.VMEM((GROUP, 128), i32),
            pltpu.VMEM((DEPTH1, 128, 64), f32),
            pltpu.VMEM_SHARED((NE2, 64), f32),
            pltpu.SemaphoreType.DMA((DEPTH1,)),
            pltpu.SemaphoreType.DMA((DEPTH1,)),
        ],
        **_SC_PARAMS,
    )(e2s, e2d, t0, t1, t2, t3, z64)


def _agg2_body(srce, dste, q, z48, out, idxg, idxd, rows, st128, st120,
               acc, sem_g, sem_s):
    cid = lax.axis_index("c")
    sid = lax.axis_index("s")
    pltpu.sync_copy(z48, st128)
    pltpu.sync_copy(z48.at[pl.ds(0, 120)], st120)
    r0 = sid * STR2
    for k in range(4):
        pltpu.sync_copy(st128, acc.at[pl.ds(r0 + k * 128, 128)])
    pltpu.sync_copy(st120, acc.at[pl.ds(r0 + 512, 120)])
    plsc.subcore_barrier()
    base = cid * 1280 + sid * 80
    _row_pipeline(srce, dste, q, acc, idxg, idxd, rows, sem_g, sem_s, base, 2,
                  DEPTH2)
    plsc.subcore_barrier()
    for k in range(4):
        pltpu.sync_copy(acc.at[pl.ds(r0 + k * 128, 128)], st128)
        pltpu.sync_copy(st128, out.at[cid, pl.ds(r0 + k * 128, 128)])
    pltpu.sync_copy(acc.at[pl.ds(r0 + 512, 120)], st120)
    pltpu.sync_copy(st120, out.at[cid, pl.ds(r0 + 512, 120)])


def _sc_agg2(srce, dste, q, z48):
    return pl.kernel(
        _agg2_body,
        out_type=jax.ShapeDtypeStruct((2, NP, 48), f32),
        mesh=_mesh(),
        scratch_types=[
            pltpu.VMEM((GROUP, 128), i32),
            pltpu.VMEM((GROUP, 128), i32),
            pltpu.VMEM((DEPTH2, 128, 48), f32),
            pltpu.VMEM((128, 48), f32),
            pltpu.VMEM((120, 48), f32),
            pltpu.VMEM_SHARED((NP, 48), f32),
            pltpu.SemaphoreType.DMA((DEPTH2,)),
            pltpu.SemaphoreType.DMA((DEPTH2,)),
        ],
        **_SC_PARAMS,
    )(srce, dste, q, z48)


def kernel(x, edge_index, is_reversed, W_st1, b_st1, W_ts1, b_ts1, W_1, b_1,
           W_2, b_2):
    src = edge_index[0].astype(i32)
    dst = edge_index[1].astype(i32)
    rev = is_reversed.astype(i32)

    padz = ((0, LENE - E),)
    e2sp, e2dp, srce, dste = _tc_indices(
        jnp.pad(src, padz).reshape(LENE // 128, 128),
        jnp.pad(dst, padz).reshape(LENE // 128, 128),
        jnp.pad(rev, padz).reshape(LENE // 128, 128))

    ones128 = jnp.ones((128,), f32)
    zh = jnp.zeros((STR1,), f32)
    z64 = jnp.zeros((128, 64), f32)
    z48 = jnp.zeros((128, 48), f32)

    wcat = jnp.concatenate([W_st1, W_1, W_ts1], axis=1)
    xp = jnp.pad(x, ((0, NP - N), (0, 0)))
    h = _tc_matmul(xp, wcat)
    h33 = h.reshape(NP, 3, 128)

    hist = _sc_hist(e2dp, ones128, zh)
    cnt2 = hist.reshape(2, NP, 2, 1)
    t0, t1, t2, t3, dis3 = _tc_tables(cnt2, h33)
    tabs = [t.reshape(NE2, 64) for t in (t0, t1, t2, t3)]

    agg = _sc_agg1(e2sp, e2dp, *tabs, z64)
    parts1 = [agg[cid, g].reshape(NP, 2, 64) for cid in range(2)
              for g in range(4)]

    bcat = jnp.stack([b_st1, b_1, b_ts1])
    w2p = jnp.pad(
        jnp.concatenate([W_2[0:128], W_2[256:384], W_2[128:256]], axis=0),
        ((0, 0), (0, 8))).reshape(3, 128, 48)

    p, q = _tc_l2(parts1, h33, dis3, bcat, w2p)

    parts = _sc_agg2(srce, dste, q, z48)

    return _tc_out(parts, p, dis3[:, 1, :], b_2.reshape(1, 40))


# speedup vs baseline: 1.0266x; 1.0266x over previous
import jax
import jax.numpy as jnp
from jax import lax
from jax.experimental import pallas as pl
from jax.experimental.pallas import tpu as pltpu
from jax.experimental.pallas import tpu_sc as plsc

f32 = jnp.float32
i32 = jnp.int32

N = 10000
E = 320000
NP = 10112
NE2 = 2 * NP
LENE = 327680
GROUP = 40
DEPTH1 = 4
DEPTH2 = 8
STR1 = NE2 // 16
STR2 = NP // 16


def _mesh():
    return plsc.VectorSubcoreMesh(
        core_axis_name="c", subcore_axis_name="s", num_cores=2,
        num_subcores=16)


_SC_PARAMS = dict(compiler_params=pltpu.CompilerParams(
    use_tc_tiling_on_sc=False))


def _mm_body(xb, wb, ob):
    ob[...] = jnp.dot(xb[...], wb[...], preferred_element_type=f32)


def _tc_matmul(xp, wcat):
    return pl.pallas_call(
        _mm_body,
        grid=(NP // 128,),
        in_specs=[pl.BlockSpec((128, 128), lambda i: (i, 0)),
                  pl.BlockSpec((128, 384), lambda i: (0, 0))],
        out_specs=pl.BlockSpec((128, 384), lambda i: (i, 0)),
        out_shape=jax.ShapeDtypeStruct((NP, 384), f32),
    )(xp, wcat)


def _idx_body(sb, db, rb, es, ed, ss, dd):
    sv, dv, rv = sb[...], db[...], rb[...]
    flat = (lax.broadcasted_iota(i32, (LENE // 128, 128), 0) * 128
            + lax.broadcasted_iota(i32, (LENE // 128, 128), 1))
    valid = flat < E
    low = lax.bitwise_and(flat, 63)
    es[...] = jnp.where(valid, sv * 2 + rv, low)
    ed[...] = jnp.where(valid, dv * 2 + rv, 2 * N + low)
    ss[...] = jnp.where(valid, sv, low)
    dd[...] = jnp.where(valid, dv, N + low)


def _tc_indices(srcm, dstm, revm):
    spec = pl.BlockSpec((LENE // 128, 128), lambda i: (0, 0))
    sh = jax.ShapeDtypeStruct((LENE // 128, 128), i32)
    return pl.pallas_call(
        _idx_body,
        grid=(1,),
        in_specs=[spec, spec, spec],
        out_specs=[spec, spec, spec, spec],
        out_shape=[sh, sh, sh, sh],
    )(srcm, dstm, revm)


def _tab_body(cb, hb, t0, t1, t2, t3, db):
    c = cb[0] + cb[1]
    cst, cts = c[:, 0, :], c[:, 1, :]
    d_st = lax.rsqrt(cst + 1.0)
    d_ts = lax.rsqrt(cts + 1.0)
    d_al = lax.rsqrt(cst + cts + 1.0)
    db[...] = jnp.concatenate(
        [d_st[:, None, :], d_al[:, None, :], d_ts[:, None, :]], axis=1)
    gst = d_st * hb[:, 0, :]
    gall = d_al * hb[:, 1, :]
    gts = d_ts * hb[:, 2, :]
    for g, tr in enumerate((t0, t1, t2, t3)):
        cg = slice(32 * g, 32 * g + 32)
        ra = jnp.concatenate([gst[:, cg], gall[:, cg]], axis=1)[:, None, :]
        rb = jnp.concatenate([gts[:, cg], gall[:, cg]], axis=1)[:, None, :]
        tr[...] = jnp.concatenate([ra, rb], axis=1)


def _tc_tables(cnt3, h33):
    sht = jax.ShapeDtypeStruct((NP, 2, 64), f32)
    return pl.pallas_call(
        _tab_body,
        grid=(NP // 128,),
        in_specs=[pl.BlockSpec((2, 128, 2, 1), lambda i: (0, i, 0, 0)),
                  pl.BlockSpec((128, 3, 128), lambda i: (i, 0, 0))],
        out_specs=[pl.BlockSpec((128, 2, 64), lambda i: (i, 0, 0))] * 4
        + [pl.BlockSpec((128, 3, 1), lambda i: (i, 0, 0))],
        out_shape=[sht] * 4 + [jax.ShapeDtypeStruct((NP, 3, 1), f32)],
    )(cnt3, h33)


def _l2_body(m00, m01, m02, m03, m10, m11, m12, m13, hb, db, bcb, wb, pb, qb):
    m0s, m1s = (m00, m01, m02, m03), (m10, m11, m12, m13)
    st, ts, al = [], [], []
    for g in range(4):
        sp = m0s[g][...] + m1s[g][...]
        st.append(sp[:, 0, 0:32])
        ts.append(sp[:, 1, 0:32])
        al.append(sp[:, 0, 32:64] + sp[:, 1, 32:64])
    aggs = (jnp.concatenate(st, axis=1), jnp.concatenate(al, axis=1),
            jnp.concatenate(ts, axis=1))
    acc = jnp.zeros((128, 48), f32)
    for c in range(3):
        dd = db[:, c, :]
        hc = jnp.maximum(
            dd * aggs[c] + dd * dd * hb[:, c, :] + bcb[c][None, :], 0.0)
        acc = acc + jnp.dot(hc, wb[c], preferred_element_type=f32)
    pb[...] = acc
    qb[...] = db[:, 1, :] * acc


def _tc_l2(parts1, h33, dis3, bcat, w2p):
    sh48 = jax.ShapeDtypeStruct((NP, 48), f32)
    return pl.pallas_call(
        _l2_body,
        grid=(NP // 128,),
        in_specs=[pl.BlockSpec((128, 2, 64), lambda i: (i, 0, 0))] * 8
        + [pl.BlockSpec((128, 3, 128), lambda i: (i, 0, 0)),
           pl.BlockSpec((128, 3, 1), lambda i: (i, 0, 0)),
           pl.BlockSpec((3, 128), lambda i: (0, 0)),
           pl.BlockSpec((3, 128, 48), lambda i: (0, 0, 0))],
        out_specs=[pl.BlockSpec((128, 48), lambda i: (i, 0)),
                   pl.BlockSpec((128, 48), lambda i: (i, 0))],
        out_shape=[sh48, sh48],
    )(*parts1, h33, dis3, bcat, w2p)


def _out_body(ptb, pb, db, b2b, ob):
    s = ptb[0] + ptb[1]
    dd = db[...]
    o = dd * s[:, :40] + dd * dd * pb[:, :40] + b2b[...]
    m = jnp.max(o, axis=1, keepdims=True)
    z = jnp.sum(jnp.exp(o - m), axis=1, keepdims=True)
    ob[...] = o - m - jnp.log(z)


def _tc_out(parts, p, disall, b2p):
    return pl.pallas_call(
        _out_body,
        grid=(NP // 128,),
        in_specs=[pl.BlockSpec((2, 128, 48), lambda i: (0, i, 0)),
                  pl.BlockSpec((128, 48), lambda i: (i, 0)),
                  pl.BlockSpec((128, 1), lambda i: (i, 0)),
                  pl.BlockSpec((1, 40), lambda i: (0, 0))],
        out_specs=pl.BlockSpec((128, 40), lambda i: (i, 0)),
        out_shape=jax.ShapeDtypeStruct((NP, 40), f32),
    )(parts, p, disall, b2p)


def _hist_body(e2d, ones_h, z_h, out, idxd, ones_v, stage, hist, sem):
    cid = lax.axis_index("c")
    sid = lax.axis_index("s")
    pltpu.sync_copy(z_h, stage)
    pltpu.sync_copy(stage, hist.at[pl.ds(sid * STR1, STR1)])
    pltpu.sync_copy(ones_h, ones_v)
    plsc.subcore_barrier()
    base = (cid * 16 + sid) * 80
    for g in range(2):
        pltpu.sync_copy(e2d.at[pl.ds(base + g * GROUP, GROUP)], idxd)

        def it_body(it, carry):
            for b in range(8):
                pltpu.async_copy(ones_v, hist.at[idxd.at[it * 8 + b]], sem,
                                 add=True)
            for b in range(8):
                pltpu.make_async_copy(ones_v, hist.at[idxd.at[0]], sem).wait()
            return carry

        lax.fori_loop(0, 5, it_body, 0)
    plsc.subcore_barrier()
    pltpu.sync_copy(hist.at[pl.ds(sid * STR1, STR1)], stage)
    pltpu.sync_copy(stage, out.at[cid, sid])


def _sc_hist(e2d, ones_h, z_h):
    return pl.kernel(
        _hist_body,
        out_type=jax.ShapeDtypeStruct((2, 16, STR1), f32),
        mesh=_mesh(),
        scratch_types=[
            pltpu.VMEM((GROUP, 128), i32),
            pltpu.VMEM((128,), f32),
            pltpu.VMEM((STR1,), f32),
            pltpu.VMEM_SHARED((NE2,), f32),
            pltpu.SemaphoreType.DMA,
        ],
        **_SC_PARAMS,
    )(e2d, ones_h, z_h)


def _row_pipeline(src_hbm, dst_hbm, table, acc, idxg, idxd, rows, sem_g,
                  sem_s, base, groups, depth):
    for g in range(groups):
        if g > 0:
            for b in range(depth):
                pltpu.make_async_copy(
                    rows.at[b], acc.at[idxd.at[0]], sem_s.at[b]).wait()
        pltpu.sync_copy(src_hbm.at[pl.ds(base + g * GROUP, GROUP)], idxg)
        pltpu.sync_copy(dst_hbm.at[pl.ds(base + g * GROUP, GROUP)], idxd)

        def it_body(it, carry):
            for b in range(depth):
                @pl.when(it > 0)
                def _drain(b=b):
                    pltpu.make_async_copy(
                        rows.at[b], acc.at[idxd.at[0]], sem_s.at[b]).wait()
                pltpu.async_copy(
                    table.at[idxg.at[it * depth + b]], rows.at[b],
                    sem_g.at[b])
            for b in range(depth):
                pltpu.make_async_copy(
                    table.at[idxg.at[0]], rows.at[b], sem_g.at[b]).wait()
                pltpu.async_copy(
                    rows.at[b], acc.at[idxd.at[it * depth + b]],
                    sem_s.at[b], add=True)
            return carry

        lax.fori_loop(0, GROUP // depth, it_body, 0)
    for b in range(depth):
        pltpu.make_async_copy(rows.at[b], acc.at[idxd.at[0]], sem_s.at[b]).wait()


def _agg1_body(e2s, e2d, t0, t1, t2, t3, z64, out, idxg, idxd, rows, acc,
               sem_g, sem_s):
    cid = lax.axis_index("c")
    sid = lax.axis_index("s")
    r0 = sid * STR1
    base = cid * 1280 + sid * 80
    for p, tbl in enumerate((t0, t1, t2, t3)):
        pltpu.sync_copy(z64, rows.at[0])
        for k in range(9):
            pltpu.sync_copy(rows.at[0], acc.at[pl.ds(r0 + k * 128, 128)])
        pltpu.sync_copy(rows.at[0, pl.ds(0, 112)],
                        acc.at[pl.ds(r0 + 1152, 112)])
        plsc.subcore_barrier()
        _row_pipeline(e2s, e2d, tbl, acc, idxg, idxd, rows, sem_g, sem_s,
                      base, 2, DEPTH1)
        plsc.subcore_barrier()
        for k in range(9):
            pltpu.sync_copy(acc.at[pl.ds(r0 + k * 128, 128)], rows.at[0])
            pltpu.sync_copy(rows.at[0], out.at[cid, p, pl.ds(r0 + k * 128, 128)])
        pltpu.sync_copy(acc.at[pl.ds(r0 + 1152, 112)],
                        rows.at[0, pl.ds(0, 112)])
        pltpu.sync_copy(rows.at[0, pl.ds(0, 112)],
                        out.at[cid, p, pl.ds(r0 + 1152, 112)])
        plsc.subcore_barrier()


def _sc_agg1(e2s, e2d, t0, t1, t2, t3, z64):
    return pl.kernel(
        _agg1_body,
        out_type=jax.ShapeDtypeStruct((2, 4, NE2, 64), f32),
        mesh=_mesh(),
        scratch_types=[
            pltpu.VMEM((GROUP, 128), i32),
            pltpu.VMEM((GROUP, 128), i32),
            pltpu.VMEM((DEPTH1, 128, 64), f32),
            pltpu.VMEM_SHARED((NE2, 64), f32),
            pltpu.SemaphoreType.DMA((DEPTH1,)),
            pltpu.SemaphoreType.DMA((DEPTH1,)),
        ],
        **_SC_PARAMS,
    )(e2s, e2d, t0, t1, t2, t3, z64)


def _agg2_body(srce, dste, q, z48, out, idxg, idxd, rows, st128, st120,
               acc, sem_g, sem_s):
    cid = lax.axis_index("c")
    sid = lax.axis_index("s")
    pltpu.sync_copy(z48, st128)
    pltpu.sync_copy(z48.at[pl.ds(0, 120)], st120)
    r0 = sid * STR2
    for k in range(4):
        pltpu.sync_copy(st128, acc.at[pl.ds(r0 + k * 128, 128)])
    pltpu.sync_copy(st120, acc.at[pl.ds(r0 + 512, 120)])
    plsc.subcore_barrier()
    base = cid * 1280 + sid * 80
    _row_pipeline(srce, dste, q, acc, idxg, idxd, rows, sem_g, sem_s, base, 2,
                  DEPTH2)
    plsc.subcore_barrier()
    for k in range(4):
        pltpu.sync_copy(acc.at[pl.ds(r0 + k * 128, 128)], st128)
        pltpu.sync_copy(st128, out.at[cid, pl.ds(r0 + k * 128, 128)])
    pltpu.sync_copy(acc.at[pl.ds(r0 + 512, 120)], st120)
    pltpu.sync_copy(st120, out.at[cid, pl.ds(r0 + 512, 120)])


def _sc_agg2(srce, dste, q, z48):
    return pl.kernel(
        _agg2_body,
        out_type=jax.ShapeDtypeStruct((2, NP, 48), f32),
        mesh=_mesh(),
        scratch_types=[
            pltpu.VMEM((GROUP, 128), i32),
            pltpu.VMEM((GROUP, 128), i32),
            pltpu.VMEM((DEPTH2, 128, 48), f32),
            pltpu.VMEM((128, 48), f32),
            pltpu.VMEM((120, 48), f32),
            pltpu.VMEM_SHARED((NP, 48), f32),
            pltpu.SemaphoreType.DMA((DEPTH2,)),
            pltpu.SemaphoreType.DMA((DEPTH2,)),
        ],
        **_SC_PARAMS,
    )(srce, dste, q, z48)


def kernel(x, edge_index, is_reversed, W_st1, b_st1, W_ts1, b_ts1, W_1, b_1,
           W_2, b_2):
    src = edge_index[0].astype(i32)
    dst = edge_index[1].astype(i32)
    rev = is_reversed.astype(i32)

    padz = ((0, LENE - E),)
    e2sp, e2dp, srce, dste = _tc_indices(
        jnp.pad(src, padz).reshape(LENE // 128, 128),
        jnp.pad(dst, padz).reshape(LENE // 128, 128),
        jnp.pad(rev, padz).reshape(LENE // 128, 128))

    ones128 = jnp.ones((128,), f32)
    zh = jnp.zeros((STR1,), f32)
    z64 = jnp.zeros((128, 64), f32)
    z48 = jnp.zeros((128, 48), f32)

    wcat = jnp.concatenate([W_st1, W_1, W_ts1], axis=1)
    xp = jnp.pad(x, ((0, NP - N), (0, 0)))
    h = _tc_matmul(xp, wcat)
    h33 = h.reshape(NP, 3, 128)

    hist = _sc_hist(e2dp, ones128, zh)
    cnt2 = hist.reshape(2, NP, 2, 1)
    t0, t1, t2, t3, dis3 = _tc_tables(cnt2, h33)
    tabs = [t.reshape(NE2, 64) for t in (t0, t1, t2, t3)]

    agg = _sc_agg1(e2sp, e2dp, *tabs, z64)
    parts1 = [agg[cid, g].reshape(NP, 2, 64) for cid in range(2)
              for g in range(4)]

    bcat = jnp.stack([b_st1, b_1, b_ts1])
    w2p = jnp.pad(
        jnp.concatenate([W_2[0:128], W_2[256:384], W_2[128:256]], axis=0),
        ((0, 0), (0, 8))).reshape(3, 128, 48)

    p, q = _tc_l2(parts1, h33, dis3, bcat, w2p)

    parts = _sc_agg2(srce, dste, q, z48)

    out = _tc_out(parts, p, dis3[:, 1, :], b_2.reshape(1, 40))
    return out[:N]
